# Initial kernel scaffold; baseline (speedup 1.0000x reference)
import jax
import jax.numpy as jnp
from jax.experimental import pallas as pl


def kernel(x, table):
    raise NotImplementedError("write your pallas kernel here")


# speedup vs baseline: 11.2468x; 11.2468x over previous
import functools

import jax
import jax.numpy as jnp
from jax import lax
from jax.experimental import pallas as pl
from jax.experimental.pallas import tpu as pltpu
from jax.experimental.pallas import tpu_sc as plsc

_VOCAB = 50257
_DIM = 2048
_BLANK = 5
_NB = 8
_B = 4
_S = 2048
_N = _B * _S
_NC = 2
_NS = 16
_NW = _NC * _NS
_PW = _N // _NW
_C = 32
_NCH = _PW // _C
_WIN = 16 + _PW


def _iota16():
    return lax.iota(jnp.int32, 16)


def _sload(ref, idx_scalar):
    v = plsc.load_gather(ref, [jnp.full((16,), idx_scalar, jnp.int32)])
    return jnp.max(v)


def _sc_body(x_hbm, table_hbm, out_hbm, xw, cw, wgt, k1a, rows, halo,
             sem_g, sem_h):
    wid = lax.axis_index("s") * _NC + lax.axis_index("c")
    base = wid * _PW
    row_start = (wid // (_S // _PW)) * _S
    is_row_first = (wid % (_S // _PW)) == 0

    xw[pl.ds(0, 16)] = jnp.zeros((16,), jnp.int32)

    @pl.when(jnp.logical_not(is_row_first))
    def _():
        pltpu.sync_copy(x_hbm.at[pl.ds(base - 16, 16)], xw.at[pl.ds(0, 16)])

    pltpu.sync_copy(x_hbm.at[pl.ds(base, _PW)], xw.at[pl.ds(16, _PW)])

    any_vec = jnp.zeros((16,), jnp.int32)
    for g in range(_WIN // 16):
        j0 = g * 16
        jv = _iota16() + j0
        xj = xw[pl.ds(j0, 16)]
        xj1 = plsc.load_gather(xw, [jnp.minimum(jv + 1, _WIN - 1)])
        pos = jv + (base - 16)
        valid = jnp.logical_and(pos >= row_start,
                                jnp.logical_and(jv >= 8, jv <= _WIN - 2))
        cbit = jnp.logical_and(jnp.logical_and(xj != _BLANK, xj1 == _BLANK),
                               valid).astype(jnp.int32)
        cw[pl.ds(j0, 16)] = cbit
        any_vec = any_vec + cbit
    worker_any = jnp.sum(any_vec)

    @pl.when(worker_any > 0)
    def _():
        for g in range(_PW // 16):
            t0 = g * 16
            tv = _iota16() + t0
            kk = plsc.load_gather(cw, [tv + 8])
            for d in range(8, 16):
                if d > 8:
                    kk = kk + plsc.load_gather(cw, [tv + d])
                m = 16 - d
                kf = kk.astype(jnp.float32)
                w = jnp.ones((16,), jnp.float32)
                for i in range(1, m + 1):
                    w = w * (kf + float(i - 1)) / float(i)
                wgt[pl.ds((m - 1) * _PW + t0, 16)] = w
            k1a[pl.ds(t0, 16)] = kk.astype(jnp.float32)

    for ci in range(_NCH):
        goff = 16 + ci * _C
        gpos = base + ci * _C

        cp = pltpu.async_copy(table_hbm.at[xw.at[pl.ds(goff, _C)]], rows,
                              sem_g)
        cp.wait()

        @pl.when(worker_any > 0)
        def _fix(ci=ci):
            ca = jnp.zeros((16,), jnp.int32)
            for q in range(3):
                ca = ca + cw[pl.ds(ci * _C + q * 16, 16)]
            chunk_any = jnp.sum(ca)

            @pl.when(chunk_any > 0)
            def _():
                hp = pltpu.async_copy(
                    table_hbm.at[xw.at[pl.ds(8 + ci * _C, 8)]], halo, sem_h)
                hp.wait()

                def pos_body(i, _):
                    t = (_C - 1) - i
                    t_abs = ci * _C + t
                    k1 = _sload(k1a, t_abs)

                    @pl.when(k1 > 0.0)
                    def _():
                        for m in range(1, _NB + 1):
                            wsc = _sload(wgt, (m - 1) * _PW + t_abs)
                            r = t - m

                            @pl.when(jnp.logical_and(wsc > 0.0, r >= 0))
                            def _(m=m, r=r, wsc=wsc):
                                wb = jnp.full((16,), wsc, jnp.float32)

                                def fma(jj, _):
                                    sl = pl.ds(jj * 16, 16)
                                    rows[t, sl] = rows[t, sl] + wb * rows[r, sl]
                                    return 0
                                lax.fori_loop(0, _DIM // 16, fma, 0)

                            @pl.when(jnp.logical_and(wsc > 0.0, r < 0))
                            def _(m=m, r=r, wsc=wsc):
                                wb = jnp.full((16,), wsc, jnp.float32)

                                def fma(jj, _):
                                    sl = pl.ds(jj * 16, 16)
                                    rows[t, sl] = (rows[t, sl]
                                                   + wb * halo[r + 8, sl])
                                    return 0
                                lax.fori_loop(0, _DIM // 16, fma, 0)
                    return 0

                lax.fori_loop(0, _C, pos_body, 0)

        pltpu.sync_copy(rows, out_hbm.at[pl.ds(gpos, _C)])


@functools.partial(jax.jit, static_argnums=())
def _run(x_flat, table):
    mesh = plsc.VectorSubcoreMesh(core_axis_name="c", subcore_axis_name="s")
    f = functools.partial(
        pl.kernel,
        out_type=jax.ShapeDtypeStruct((_N, _DIM), jnp.float32),
        mesh=mesh,
        compiler_params=pltpu.CompilerParams(needs_layout_passes=False),
        scratch_types=[
            pltpu.VMEM((_WIN,), jnp.int32),
            pltpu.VMEM((_WIN,), jnp.int32),
            pltpu.VMEM((_NB * _PW,), jnp.float32),
            pltpu.VMEM((_PW,), jnp.float32),
            pltpu.VMEM((_C, _DIM), jnp.float32),
            pltpu.VMEM((_NB, _DIM), jnp.float32),
            pltpu.SemaphoreType.DMA,
            pltpu.SemaphoreType.DMA,
        ],
    )(_sc_body)
    return f(x_flat, table)


def kernel(x, table):
    out = _run(x.reshape(_N), table)
    return out.reshape(_B, _S, _DIM)


# speedup vs baseline: 12.9044x; 1.1474x over previous
import functools

import jax
import jax.numpy as jnp
from jax import lax
from jax.experimental import pallas as pl
from jax.experimental.pallas import tpu as pltpu
from jax.experimental.pallas import tpu_sc as plsc

_VOCAB = 50257
_DIM = 2048
_BLANK = 5
_NB = 8
_B = 4
_S = 2048
_N = _B * _S
_NC = 2
_NS = 16
_NW = _NC * _NS
_PW = _N // _NW
_C = 16
_NCH = _PW // _C
_WIN = 16 + _PW


def _iota16():
    return lax.iota(jnp.int32, 16)


def _sload(ref, idx_scalar):
    v = plsc.load_gather(ref, [jnp.full((16,), idx_scalar, jnp.int32)])
    return jnp.max(v)


def _sc_body(x_hbm, table_hbm, out_hbm, xw, cw, wgt, k1a, rows0, rows1,
             halo, sem_g0, sem_g1, sem_s0, sem_s1, sem_h):
    wid = lax.axis_index("s") * _NC + lax.axis_index("c")
    base = wid * _PW
    row_start = (wid // (_S // _PW)) * _S
    is_row_first = (wid % (_S // _PW)) == 0

    xw[pl.ds(0, 16)] = jnp.zeros((16,), jnp.int32)

    @pl.when(jnp.logical_not(is_row_first))
    def _():
        pltpu.sync_copy(x_hbm.at[pl.ds(base - 16, 16)], xw.at[pl.ds(0, 16)])

    pltpu.sync_copy(x_hbm.at[pl.ds(base, _PW)], xw.at[pl.ds(16, _PW)])

    def cbody(g, acc):
        j0 = g * 16
        jv = _iota16() + j0
        xj = xw[pl.ds(j0, 16)]
        xj1 = plsc.load_gather(xw, [jnp.minimum(jv + 1, _WIN - 1)])
        pos = jv + (base - 16)
        valid = jnp.logical_and(pos >= row_start,
                                jnp.logical_and(jv >= 8, jv <= _WIN - 2))
        cbit = jnp.logical_and(jnp.logical_and(xj != _BLANK, xj1 == _BLANK),
                               valid).astype(jnp.int32)
        cw[pl.ds(j0, 16)] = cbit
        return acc + cbit

    any_vec = lax.fori_loop(0, _WIN // 16, cbody, jnp.zeros((16,), jnp.int32))
    worker_any = jnp.sum(any_vec)

    @pl.when(worker_any > 0)
    def _():
        def wbody(g, _):
            t0 = g * 16
            tv = _iota16() + t0
            kk = plsc.load_gather(cw, [tv + 8])
            for d in range(8, 16):
                if d > 8:
                    kk = kk + plsc.load_gather(cw, [tv + d])
                m = 16 - d
                kf = kk.astype(jnp.float32)
                w = jnp.ones((16,), jnp.float32)
                for i in range(1, m + 1):
                    w = w * (kf + float(i - 1)) / float(i)
                wgt[pl.ds((m - 1) * _PW + t0, 16)] = w
            k1a[pl.ds(t0, 16)] = kk.astype(jnp.float32)
            return 0

        lax.fori_loop(0, _PW // 16, wbody, 0)

    def fixup(ci, rows):
        ca = (cw[pl.ds(ci * _C, 16)] + cw[pl.ds(ci * _C + 16, 16)])
        chunk_any = jnp.sum(ca)

        @pl.when(chunk_any > 0)
        def _():
            hp = pltpu.async_copy(
                table_hbm.at[xw.at[pl.ds(8 + ci * _C, 8)]], halo, sem_h)
            hp.wait()

            def pos_body(i, _):
                t = (_C - 1) - i
                t_abs = ci * _C + t
                k1 = _sload(k1a, t_abs)

                @pl.when(k1 > 0.0)
                def _():
                    for m in range(1, _NB + 1):
                        wsc = _sload(wgt, (m - 1) * _PW + t_abs)
                        r = t - m

                        @pl.when(jnp.logical_and(wsc > 0.0, r >= 0))
                        def _(m=m, r=r, wsc=wsc):
                            wb = jnp.full((16,), wsc, jnp.float32)

                            def fma(jj, _):
                                sl = pl.ds(jj * 16, 16)
                                rows[t, sl] = rows[t, sl] + wb * rows[r, sl]
                                return 0
                            lax.fori_loop(0, _DIM // 16, fma, 0)

                        @pl.when(jnp.logical_and(wsc > 0.0, r < 0))
                        def _(m=m, r=r, wsc=wsc):
                            wb = jnp.full((16,), wsc, jnp.float32)

                            def fma(jj, _):
                                sl = pl.ds(jj * 16, 16)
                                rows[t, sl] = (rows[t, sl]
                                               + wb * halo[r + 8, sl])
                                return 0
                            lax.fori_loop(0, _DIM // 16, fma, 0)
                return 0

            lax.fori_loop(0, _C, pos_body, 0)

    bufs = (rows0, rows1)
    gsems = (sem_g0, sem_g1)
    ssems = (sem_s0, sem_s1)

    def gather_copy(ci, b):
        return pltpu.make_async_copy(
            table_hbm.at[xw.at[pl.ds(16 + ci * _C, _C)]], bufs[b], gsems[b])

    def scatter_copy(ci, b):
        return pltpu.make_async_copy(
            bufs[b], out_hbm.at[pl.ds(base + ci * _C, _C)], ssems[b])

    gather_copy(0, 0).start()

    @pl.loop(0, _NCH, step=2)
    def _chunks(g):
        for b in range(2):
            ci = g + b
            nb = 1 - b
            gather_copy(ci, b).wait()

            @pl.when(ci + 1 < _NCH)
            def _(ci=ci, nb=nb):
                @pl.when(ci >= 1)
                def _():
                    scatter_copy(ci - 1, nb).wait()
                gather_copy(ci + 1, nb).start()

            @pl.when(worker_any > 0)
            def _(ci=ci, b=b):
                fixup(ci, bufs[b])

            scatter_copy(ci, b).start()

    scatter_copy(_NCH - 2, 0).wait()
    scatter_copy(_NCH - 1, 1).wait()


@functools.partial(jax.jit, static_argnums=())
def _run(x_flat, table):
    mesh = plsc.VectorSubcoreMesh(core_axis_name="c", subcore_axis_name="s")
    f = functools.partial(
        pl.kernel,
        out_type=jax.ShapeDtypeStruct((_N, _DIM), jnp.float32),
        mesh=mesh,
        compiler_params=pltpu.CompilerParams(needs_layout_passes=False),
        scratch_types=[
            pltpu.VMEM((_WIN,), jnp.int32),
            pltpu.VMEM((_WIN,), jnp.int32),
            pltpu.VMEM((_NB * _PW,), jnp.float32),
            pltpu.VMEM((_PW,), jnp.float32),
            pltpu.VMEM((_C, _DIM), jnp.float32),
            pltpu.VMEM((_C, _DIM), jnp.float32),
            pltpu.VMEM((_NB, _DIM), jnp.float32),
            pltpu.SemaphoreType.DMA,
            pltpu.SemaphoreType.DMA,
            pltpu.SemaphoreType.DMA,
            pltpu.SemaphoreType.DMA,
            pltpu.SemaphoreType.DMA,
        ],
    )(_sc_body)
    return f(x_flat, table)


def kernel(x, table):
    out = _run(x.reshape(_N), table)
    return out.reshape(_B, _S, _DIM)


# speedup vs baseline: 13.0103x; 1.0082x over previous
import functools

import jax
import jax.numpy as jnp
from jax import lax
from jax.experimental import pallas as pl
from jax.experimental.pallas import tpu as pltpu
from jax.experimental.pallas import tpu_sc as plsc

_VOCAB = 50257
_DIM = 2048
_BLANK = 5
_NB = 8
_B = 4
_S = 2048
_N = _B * _S
_NC = 2
_NS = 16
_NW = _NC * _NS
_PW = _N // _NW
_C = 16
_NCH = _PW // _C
_WIN = 16 + _PW


def _iota16():
    return lax.iota(jnp.int32, 16)


def _sload(ref, idx_scalar):
    v = plsc.load_gather(ref, [jnp.full((16,), idx_scalar, jnp.int32)])
    return jnp.max(v)


def _sc_body(x_hbm, table_hbm, out_hbm, xw, cw, wgt, k1a, rows0, rows1,
             rows2, halo, sem_g0, sem_g1, sem_g2, sem_s0, sem_s1, sem_s2,
             sem_h):
    wid = lax.axis_index("s") * _NC + lax.axis_index("c")
    base = wid * _PW
    row_start = (wid // (_S // _PW)) * _S
    is_row_first = (wid % (_S // _PW)) == 0

    xw[pl.ds(0, 16)] = jnp.zeros((16,), jnp.int32)

    @pl.when(jnp.logical_not(is_row_first))
    def _():
        pltpu.sync_copy(x_hbm.at[pl.ds(base - 16, 16)], xw.at[pl.ds(0, 16)])

    pltpu.sync_copy(x_hbm.at[pl.ds(base, _PW)], xw.at[pl.ds(16, _PW)])

    def cbody(g, acc):
        j0 = g * 16
        jv = _iota16() + j0
        xj = xw[pl.ds(j0, 16)]
        xj1 = plsc.load_gather(xw, [jnp.minimum(jv + 1, _WIN - 1)])
        pos = jv + (base - 16)
        valid = jnp.logical_and(pos >= row_start,
                                jnp.logical_and(jv >= 8, jv <= _WIN - 2))
        cbit = jnp.logical_and(jnp.logical_and(xj != _BLANK, xj1 == _BLANK),
                               valid).astype(jnp.int32)
        cw[pl.ds(j0, 16)] = cbit
        return acc + cbit

    any_vec = lax.fori_loop(0, _WIN // 16, cbody, jnp.zeros((16,), jnp.int32))
    worker_any = jnp.sum(any_vec)

    @pl.when(worker_any > 0)
    def _():
        def wbody(g, _):
            t0 = g * 16
            tv = _iota16() + t0
            kk = plsc.load_gather(cw, [tv + 8])
            for d in range(8, 16):
                if d > 8:
                    kk = kk + plsc.load_gather(cw, [tv + d])
                m = 16 - d
                kf = kk.astype(jnp.float32)
                w = jnp.ones((16,), jnp.float32)
                for i in range(1, m + 1):
                    w = w * (kf + float(i - 1)) / float(i)
                wgt[pl.ds((m - 1) * _PW + t0, 16)] = w
            k1a[pl.ds(t0, 16)] = kk.astype(jnp.float32)
            return 0

        lax.fori_loop(0, _PW // 16, wbody, 0)

    def fixup(ci, rows):
        ca = (cw[pl.ds(ci * _C, 16)] + cw[pl.ds(ci * _C + 16, 16)])
        chunk_any = jnp.sum(ca)

        @pl.when(chunk_any > 0)
        def _():
            hp = pltpu.async_copy(
                table_hbm.at[xw.at[pl.ds(8 + ci * _C, 8)]], halo, sem_h)
            hp.wait()

            def pos_body(i, _):
                t = (_C - 1) - i
                t_abs = ci * _C + t
                k1 = _sload(k1a, t_abs)

                @pl.when(k1 > 0.0)
                def _():
                    for m in range(1, _NB + 1):
                        wsc = _sload(wgt, (m - 1) * _PW + t_abs)
                        r = t - m

                        @pl.when(jnp.logical_and(wsc > 0.0, r >= 0))
                        def _(m=m, r=r, wsc=wsc):
                            wb = jnp.full((16,), wsc, jnp.float32)

                            def fma(jj, _):
                                sl = pl.ds(jj * 16, 16)
                                rows[t, sl] = rows[t, sl] + wb * rows[r, sl]
                                return 0
                            lax.fori_loop(0, _DIM // 16, fma, 0)

                        @pl.when(jnp.logical_and(wsc > 0.0, r < 0))
                        def _(m=m, r=r, wsc=wsc):
                            wb = jnp.full((16,), wsc, jnp.float32)

                            def fma(jj, _):
                                sl = pl.ds(jj * 16, 16)
                                rows[t, sl] = (rows[t, sl]
                                               + wb * halo[r + 8, sl])
                                return 0
                            lax.fori_loop(0, _DIM // 16, fma, 0)
                return 0

            lax.fori_loop(0, _C, pos_body, 0)

    bufs = (rows0, rows1, rows2)
    gsems = (sem_g0, sem_g1, sem_g2)
    ssems = (sem_s0, sem_s1, sem_s2)

    def gather_copy(ci, b):
        return pltpu.make_async_copy(
            table_hbm.at[xw.at[pl.ds(16 + ci * _C, _C)]], bufs[b], gsems[b])

    def scatter_copy(ci, b):
        return pltpu.make_async_copy(
            bufs[b], out_hbm.at[pl.ds(base + ci * _C, _C)], ssems[b])

    def step(ci, b):
        nb = (b + 2) % 3
        gather_copy(ci, b).wait()

        @pl.when(ci + 2 < _NCH)
        def _():
            @pl.when(ci >= 1)
            def _():
                scatter_copy(ci - 1, nb).wait()
            gather_copy(ci + 2, nb).start()

        @pl.when(worker_any > 0)
        def _():
            fixup(ci, bufs[b])

        scatter_copy(ci, b).start()

    gather_copy(0, 0).start()
    gather_copy(1, 1).start()

    @pl.loop(0, _NCH - 1, step=3)
    def _chunks(g):
        for b in range(3):
            step(g + b, b)

    step(_NCH - 1, (_NCH - 1) % 3)
    scatter_copy(_NCH - 3, (_NCH - 3) % 3).wait()
    scatter_copy(_NCH - 2, (_NCH - 2) % 3).wait()
    scatter_copy(_NCH - 1, (_NCH - 1) % 3).wait()


@functools.partial(jax.jit, static_argnums=())
def _run(x_flat, table):
    mesh = plsc.VectorSubcoreMesh(core_axis_name="c", subcore_axis_name="s")
    f = functools.partial(
        pl.kernel,
        out_type=jax.ShapeDtypeStruct((_N, _DIM), jnp.float32),
        mesh=mesh,
        compiler_params=pltpu.CompilerParams(needs_layout_passes=False),
        scratch_types=[
            pltpu.VMEM((_WIN,), jnp.int32),
            pltpu.VMEM((_WIN,), jnp.int32),
            pltpu.VMEM((_NB * _PW,), jnp.float32),
            pltpu.VMEM((_PW,), jnp.float32),
            pltpu.VMEM((_C, _DIM), jnp.float32),
            pltpu.VMEM((_C, _DIM), jnp.float32),
            pltpu.VMEM((_C, _DIM), jnp.float32),
            pltpu.VMEM((_NB, _DIM), jnp.float32),
            pltpu.SemaphoreType.DMA,
            pltpu.SemaphoreType.DMA,
            pltpu.SemaphoreType.DMA,
            pltpu.SemaphoreType.DMA,
            pltpu.SemaphoreType.DMA,
            pltpu.SemaphoreType.DMA,
            pltpu.SemaphoreType.DMA,
        ],
    )(_sc_body)
    return f(x_flat, table)


def kernel(x, table):
    out = _run(x.reshape(_N), table)
    return out.reshape(_B, _S, _DIM)
